# baseline (speedup 1.0000x reference)
import functools

import jax
import jax.numpy as jnp
from jax import lax
from jax.experimental import pallas as pl
from jax.experimental.pallas import tpu as pltpu
from jax.experimental.pallas import tpu_sc as plsc

C = 16
RES = 512
NC = 2
NS = 16
NW = NC * NS
B = 256
L = 16
NG = 6 * B
PLANE_DIMS = ((1, 2), (0, 2), (0, 1))
MASK_HI = -65536


def _floor_parts(coord):
    p = coord * RES - 0.5
    t = p.astype(jnp.int32)
    tf = t.astype(jnp.float32)
    neg = tf > p
    fl_i = jnp.where(neg, t - 1, t)
    fl_f = jnp.where(neg, tf - 1.0, tf)
    w = p - fl_f
    i0 = jnp.clip(fl_i, 0, RES - 1)
    i1 = jnp.minimum(i0 + 1, RES - 1)
    return i0, i1, w


def _splat(vec, lane):
    idx = jnp.full((L,), lane, jnp.int32)
    return jnp.take_along_axis(vec, idx, axis=0)


def _sc_body(n_pad, x_hbm, fm_hbm, out_hbm, xb, idxb, wb, rows, outb, sems):
    per_w = n_pad // NW
    n_chunks = per_w // B
    wid = lax.axis_index("s") * NC + lax.axis_index("c")
    fm2 = fm_hbm

    def stage_and_index(k, par):
        base = wid * per_w + k * B
        for d in range(3):
            pltpu.sync_copy(
                x_hbm.at[pl.ds(d * n_pad + base, B)], xb.at[pl.ds(d * B, B)]
            )
        io = par * NG
        wo = par * (6 * B)

        @plsc.parallel_loop(0, B // L)
        def idx_body(g):
            o = g * L
            coords = [xb[pl.ds(d * B + o, L)] for d in range(3)]
            for plane, (ud, vd) in enumerate(PLANE_DIMS):
                x0, _, wx = _floor_parts(coords[ud])
                y0, y1, wy = _floor_parts(coords[vd])
                pbase = plane * RES * RES
                idxb[pl.ds(io + (2 * plane + 0) * B + o, L)] = (
                    pbase + (y0 << 9) + x0
                )
                idxb[pl.ds(io + (2 * plane + 1) * B + o, L)] = (
                    pbase + (y1 << 9) + x0
                )
                wb[pl.ds(wo + (2 * plane + 0) * B + o, L)] = wx
                wb[pl.ds(wo + (2 * plane + 1) * B + o, L)] = wy

        for h in range(NG // 128):
            pltpu.async_copy(
                fm2.at[idxb.at[pl.ds(io + h * 128, 128)]],
                rows.at[pl.ds(par * NG + h * 128, 128)],
                sems.at[par],
            )

    def wait_gathers(par):
        for h in range(NG // 128):
            pltpu.make_async_copy(
                fm2.at[idxb.at[pl.ds(par * NG + h * 128, 128)]],
                rows.at[pl.ds(par * NG + h * 128, 128)],
                sems.at[par],
            ).wait()

    def blend_and_store(k, par):
        base = wid * per_w + k * B
        ro = par * NG
        wo = par * (6 * B)

        @plsc.parallel_loop(0, B // L, unroll=2)
        def blend_body(g):
            o = g * L
            wv = [wb[pl.ds(wo + i * B + o, L)] for i in range(6)]
            for p in range(L):
                pt = o + p
                for plane in range(3):
                    wx = _splat(wv[2 * plane + 0], p)
                    wy = _splat(wv[2 * plane + 1], p)
                    pr0 = rows[ro + (2 * plane + 0) * B + pt, :]
                    pr1 = rows[ro + (2 * plane + 1) * B + pt, :]
                    f00 = lax.bitcast_convert_type(pr0 << 16, jnp.float32)
                    f01 = lax.bitcast_convert_type(pr0 & MASK_HI, jnp.float32)
                    f10 = lax.bitcast_convert_type(pr1 << 16, jnp.float32)
                    f11 = lax.bitcast_convert_type(pr1 & MASK_HI, jnp.float32)
                    top = f00 + wx * (f01 - f00)
                    bot = f10 + wx * (f11 - f10)
                    outb[pl.ds(pt * 3 * C + plane * C, C)] = (
                        top + wy * (bot - top)
                    )

        pltpu.sync_copy(outb, out_hbm.at[pl.ds(base * 3 * C, B * 3 * C)])

    assert n_chunks % 2 == 1
    stage_and_index(0, 0)

    def pair_body(j, _):
        k = 2 * j
        stage_and_index(k + 1, 1)
        wait_gathers(0)
        blend_and_store(k, 0)
        stage_and_index(k + 2, 0)
        wait_gathers(1)
        blend_and_store(k + 1, 1)
        return ()

    lax.fori_loop(0, (n_chunks - 1) // 2, pair_body, ())
    wait_gathers(0)
    blend_and_store(n_chunks - 1, 0)


@jax.jit
def kernel(x, fm):
    n = x.shape[0]
    per_w = -(-n // (NW * B)) * B
    n_pad = per_w * NW
    x_pad = jnp.zeros((3, n_pad), jnp.float32).at[:, :n].set(x.T).reshape(-1)
    fm_sh = jnp.concatenate([fm[:, :, 1:, :], fm[:, :, -1:, :]], axis=2)
    fm_pairs = lax.bitcast_convert_type(
        jnp.stack([fm, fm_sh], axis=-1).astype(jnp.bfloat16), jnp.int32
    ).reshape(3 * RES * RES, C)

    mesh = plsc.VectorSubcoreMesh(
        core_axis_name="c", subcore_axis_name="s", num_cores=NC, num_subcores=NS
    )
    out = pl.kernel(
        functools.partial(_sc_body, n_pad),
        out_type=jax.ShapeDtypeStruct((n_pad * 3 * C,), jnp.float32),
        mesh=mesh,
        scratch_types=[
            pltpu.VMEM((3 * B,), jnp.float32),
            pltpu.VMEM((2 * NG,), jnp.int32),
            pltpu.VMEM((2 * 6 * B,), jnp.float32),
            pltpu.VMEM((2 * NG, C), jnp.int32),
            pltpu.VMEM((B * 3 * C,), jnp.float32),
            pltpu.SemaphoreType.DMA((2,)),
        ],
        compiler_params=pltpu.CompilerParams(use_tc_tiling_on_sc=False),
    )(x_pad, fm_pairs)
    return out[: n * 3 * C].reshape(n, 3 * C)


# speedup vs baseline: 1.1041x; 1.1041x over previous
import functools

import jax
import jax.numpy as jnp
from jax import lax
from jax.experimental import pallas as pl
from jax.experimental.pallas import tpu as pltpu
from jax.experimental.pallas import tpu_sc as plsc

C = 16
RES = 512
NC = 2
NS = 16
NW = NC * NS
B = 256
L = 16
NG = 6 * B
PLANE_DIMS = ((1, 2), (0, 2), (0, 1))
MASK_HI = -65536


def _floor_parts(coord):
    p = coord * RES - 0.5
    t = p.astype(jnp.int32)
    tf = t.astype(jnp.float32)
    neg = tf > p
    fl_i = jnp.where(neg, t - 1, t)
    fl_f = jnp.where(neg, tf - 1.0, tf)
    w = p - fl_f
    i0 = jnp.clip(fl_i, 0, RES - 1)
    i1 = jnp.minimum(i0 + 1, RES - 1)
    return i0, i1, w


def _splat(vec, lane):
    idx = jnp.full((L,), lane, jnp.int32)
    return jnp.take_along_axis(vec, idx, axis=0)


def _sc_body(n_pad, x_hbm, fm_hbm, out_hbm, xb, idxb, wb, rows, outb, sems, xsem, osems):
    per_w = n_pad // NW
    n_chunks = per_w // B
    wid = lax.axis_index("s") * NC + lax.axis_index("c")
    fm2 = fm_hbm

    def stage_and_index(k, par):
        base = wid * per_w + k * B
        xcopies = [
            pltpu.async_copy(
                x_hbm.at[pl.ds(d * n_pad + base, B)],
                xb.at[pl.ds(d * B, B)],
                xsem,
            )
            for d in range(3)
        ]
        for cp in xcopies:
            cp.wait()
        io = par * NG
        wo = par * (6 * B)

        @plsc.parallel_loop(0, B // L)
        def idx_body(g):
            o = g * L
            coords = [xb[pl.ds(d * B + o, L)] for d in range(3)]
            for plane, (ud, vd) in enumerate(PLANE_DIMS):
                x0, _, wx = _floor_parts(coords[ud])
                y0, y1, wy = _floor_parts(coords[vd])
                pbase = plane * RES * RES
                idxb[pl.ds(io + (2 * plane + 0) * B + o, L)] = (
                    pbase + (y0 << 9) + x0
                )
                idxb[pl.ds(io + (2 * plane + 1) * B + o, L)] = (
                    pbase + (y1 << 9) + x0
                )
                wb[pl.ds(wo + (2 * plane + 0) * B + o, L)] = wx
                wb[pl.ds(wo + (2 * plane + 1) * B + o, L)] = wy

        for h in range(NG // 128):
            pltpu.async_copy(
                fm2.at[idxb.at[pl.ds(io + h * 128, 128)]],
                rows.at[pl.ds(par * NG + h * 128, 128)],
                sems.at[par],
            )

    def wait_gathers(par):
        for h in range(NG // 128):
            pltpu.make_async_copy(
                fm2.at[idxb.at[pl.ds(par * NG + h * 128, 128)]],
                rows.at[pl.ds(par * NG + h * 128, 128)],
                sems.at[par],
            ).wait()

    def out_copy(k, par):
        base = wid * per_w + k * B
        return pltpu.make_async_copy(
            outb.at[pl.ds(par * B * 3 * C, B * 3 * C)],
            out_hbm.at[pl.ds(base * 3 * C, B * 3 * C)],
            osems.at[par],
        )

    def blend_and_store(k, par):
        base = wid * per_w + k * B
        ro = par * NG
        wo = par * (6 * B)
        oo = par * B * 3 * C

        @pl.when(k >= 2)
        def _():
            out_copy(k - 2, par).wait()

        @plsc.parallel_loop(0, B // L)
        def blend_body(g):
            o = g * L
            wv = [wb[pl.ds(wo + i * B + o, L)] for i in range(6)]
            for p in range(L):
                pt = o + p
                for plane in range(3):
                    wx = _splat(wv[2 * plane + 0], p)
                    wy = _splat(wv[2 * plane + 1], p)
                    pr0 = rows[ro + (2 * plane + 0) * B + pt, :]
                    pr1 = rows[ro + (2 * plane + 1) * B + pt, :]
                    f00 = lax.bitcast_convert_type(pr0 << 16, jnp.float32)
                    f01 = lax.bitcast_convert_type(pr0 & MASK_HI, jnp.float32)
                    f10 = lax.bitcast_convert_type(pr1 << 16, jnp.float32)
                    f11 = lax.bitcast_convert_type(pr1 & MASK_HI, jnp.float32)
                    top = f00 + wx * (f01 - f00)
                    bot = f10 + wx * (f11 - f10)
                    outb[pl.ds(oo + pt * 3 * C + plane * C, C)] = (
                        top + wy * (bot - top)
                    )

        out_copy(k, par).start()

    assert n_chunks % 2 == 1
    stage_and_index(0, 0)

    def pair_body(j, _):
        k = 2 * j
        stage_and_index(k + 1, 1)
        wait_gathers(0)
        blend_and_store(k, 0)
        stage_and_index(k + 2, 0)
        wait_gathers(1)
        blend_and_store(k + 1, 1)
        return ()

    lax.fori_loop(0, (n_chunks - 1) // 2, pair_body, ())
    wait_gathers(0)
    blend_and_store(n_chunks - 1, 0)
    out_copy(n_chunks - 2, 1).wait()
    out_copy(n_chunks - 1, 0).wait()


@jax.jit
def kernel(x, fm):
    n = x.shape[0]
    per_w = -(-n // (NW * B)) * B
    n_pad = per_w * NW
    x_pad = jnp.zeros((3, n_pad), jnp.float32).at[:, :n].set(x.T).reshape(-1)
    fm_sh = jnp.concatenate([fm[:, :, 1:, :], fm[:, :, -1:, :]], axis=2)
    fm_pairs = lax.bitcast_convert_type(
        jnp.stack([fm, fm_sh], axis=-1).astype(jnp.bfloat16), jnp.int32
    ).reshape(3 * RES * RES, C)

    mesh = plsc.VectorSubcoreMesh(
        core_axis_name="c", subcore_axis_name="s", num_cores=NC, num_subcores=NS
    )
    out = pl.kernel(
        functools.partial(_sc_body, n_pad),
        out_type=jax.ShapeDtypeStruct((n_pad * 3 * C,), jnp.float32),
        mesh=mesh,
        scratch_types=[
            pltpu.VMEM((3 * B,), jnp.float32),
            pltpu.VMEM((2 * NG,), jnp.int32),
            pltpu.VMEM((2 * 6 * B,), jnp.float32),
            pltpu.VMEM((2 * NG, C), jnp.int32),
            pltpu.VMEM((2 * B * 3 * C,), jnp.float32),
            pltpu.SemaphoreType.DMA((2,)),
            pltpu.SemaphoreType.DMA,
            pltpu.SemaphoreType.DMA((2,)),
        ],
        compiler_params=pltpu.CompilerParams(use_tc_tiling_on_sc=False),
    )(x_pad, fm_pairs)
    return out[: n * 3 * C].reshape(n, 3 * C)
